# baseline (speedup 1.0000x reference)
import jax
import jax.numpy as jnp
from jax.experimental import pallas as pl
from jax.experimental.pallas import tpu as pltpu

N_PERIOD = 4
N_STATIONS = 162
N_PLATFORMS = 460
N_CONT = 7
N_FEAT = 12
LANES = 128
J = 32
GROUP = J * LANES

ROW_PLAT = 0
ROW_STAT = 4
N_TAB_ROWS = 16


def _bl_kernel(x_ref, y_ref, tab_ref, wc_ref, tp_ref, wy_ref, by_ref,
               out_ref, scr):
    bsg = x_ref.shape[0]
    i32 = jnp.int32
    f32 = jnp.float32

    def gather_row(r, idx):
        row = jnp.broadcast_to(tab_ref[r:r + 1, :], (J, LANES))
        return jnp.take_along_axis(row, idx, axis=1, mode="promise_in_bounds")

    for g in range(bsg):
        xt = jnp.transpose(x_ref[g], (1, 0))
        scr[g * N_FEAT * J:(g + 1) * N_FEAT * J] = xt

    for g in range(bsg):
        base = g * N_FEAT * J

        def feat(k):
            return scr[base + k:base + N_FEAT * J:N_FEAT, :]

        idx_p = feat(0).astype(i32)
        acc = jnp.full((J, LANES), tp_ref[0], f32)
        for r in range(1, N_PERIOD):
            acc = jnp.where(idx_p == r, tp_ref[r], acc)

        for s in range(3):
            idx = feat(1 + s).astype(i32)
            lo = jnp.bitwise_and(idx, LANES - 1)
            g0 = gather_row(ROW_STAT + 2 * s, lo)
            g1 = gather_row(ROW_STAT + 2 * s + 1, lo)
            acc = acc + jnp.where(idx < LANES, g0, g1)

        idx = feat(4).astype(i32)
        lo = jnp.bitwise_and(idx, LANES - 1)
        hi = jnp.right_shift(idx, 7)
        gp = gather_row(ROW_PLAT, lo)
        for c in range(1, 4):
            gp = jnp.where(hi == c, gather_row(ROW_PLAT + c, lo), gp)
        acc = acc + gp

        cont = jnp.dot(x_ref[g], wc_ref[...], preferred_element_type=f32)

        out_ref[g] = (cont + jnp.transpose(acc, (1, 0))
                      + y_ref[g] * wy_ref[0] + by_ref[0])


def kernel(emb_period, emb_stations, emb_platforms, w_period, w_stations,
           w_platforms, w_fcn, w_fcy, b_fcy, x, y):
    if x.ndim == 1:
        x = x.reshape(1, -1)
    n = x.shape[0]
    bsg = 8
    blk = bsg * GROUP
    n_pad = ((n + blk - 1) // blk) * blk
    n_grp = n_pad // GROUP
    f32 = jnp.float32

    x = x.astype(f32)
    yv = y.reshape(n).astype(f32)
    if n_pad != n:
        x = jnp.pad(x, ((0, n_pad - n), (0, 0)))
        yv = jnp.pad(yv, (0, n_pad - n))
    x3 = x.reshape(n_grp, LANES, N_FEAT * J)
    y3 = yv.reshape(n_grp, LANES, J)

    tp = (emb_period @ w_period.T).reshape(N_PERIOD)
    ts = jnp.stack(
        [(emb_stations @ w_stations[:, 3 * c:3 * c + 3].T)[:, 0]
         for c in range(3)], axis=0)
    tpl = (emb_platforms @ w_platforms.T).reshape(N_PLATFORMS)

    tab = jnp.zeros((N_TAB_ROWS, LANES), f32)
    tab = tab.at[ROW_PLAT:ROW_PLAT + 4, :].set(
        jnp.pad(tpl, (0, 4 * LANES - N_PLATFORMS)).reshape(4, LANES))
    tab = tab.at[ROW_STAT:ROW_STAT + 6, :].set(
        jnp.pad(ts, ((0, 0), (0, 2 * LANES - N_STATIONS))).reshape(6, LANES))

    wn = w_fcn.reshape(N_CONT).astype(f32)
    wc = jnp.zeros((N_FEAT * J, J), f32)
    jj = jnp.arange(J)
    for k in range(N_CONT):
        wc = wc.at[jj * N_FEAT + 5 + k, jj].set(wn[k])

    wy = w_fcy.reshape(1).astype(f32)
    by = b_fcy.reshape(1).astype(f32)

    smem = pl.BlockSpec(memory_space=pltpu.MemorySpace.SMEM)
    grid = (n_pad // blk,)

    out = pl.pallas_call(
        _bl_kernel,
        out_shape=jax.ShapeDtypeStruct((n_grp, LANES, J), f32),
        grid=grid,
        in_specs=[
            pl.BlockSpec((bsg, LANES, N_FEAT * J), lambda i: (i, 0, 0)),
            pl.BlockSpec((bsg, LANES, J), lambda i: (i, 0, 0)),
            pl.BlockSpec((N_TAB_ROWS, LANES), lambda i: (0, 0)),
            pl.BlockSpec((N_FEAT * J, J), lambda i: (0, 0)),
            smem,
            smem,
            smem,
        ],
        out_specs=pl.BlockSpec((bsg, LANES, J), lambda i: (i, 0, 0)),
        scratch_shapes=[pltpu.VMEM((bsg * J * N_FEAT, LANES), f32)],
        compiler_params=pltpu.CompilerParams(
            dimension_semantics=("parallel",),
            vmem_limit_bytes=64 * 1024 * 1024),
    )(x3, y3, tab, wc, tp.astype(f32), wy, by)

    return out.reshape(-1)[:n].reshape(n, 1)


# speedup vs baseline: 1.0407x; 1.0407x over previous
import jax
import jax.numpy as jnp
from jax.experimental import pallas as pl
from jax.experimental.pallas import tpu as pltpu

N_PERIOD = 4
N_STATIONS = 162
N_PLATFORMS = 460
N_CONT = 7
N_FEAT = 12
LANES = 128
J = 128
GROUP = J * LANES

ROW_PLAT = 0
ROW_STAT = 4
N_TAB_ROWS = 16


def _bl_kernel(x_ref, y_ref, tab_ref, tp_ref, wn_ref, wy_ref, by_ref,
               out_ref, scr):
    bsg = x_ref.shape[0]
    i32 = jnp.int32
    f32 = jnp.float32

    def gather_row(r, idx):
        row = jnp.broadcast_to(tab_ref[r:r + 1, :], (J, LANES))
        return jnp.take_along_axis(row, idx, axis=1, mode="promise_in_bounds")

    for g in range(bsg):
        xt = jnp.transpose(x_ref[g], (1, 0))
        scr[g * N_FEAT * J:(g + 1) * N_FEAT * J] = xt

    for g in range(bsg):
        base = g * N_FEAT * J

        def feat(k):
            return scr[base + k:base + N_FEAT * J:N_FEAT, :]

        idx_p = feat(0).astype(i32)
        acc = jnp.full((J, LANES), tp_ref[0], f32)
        for r in range(1, N_PERIOD):
            acc = jnp.where(idx_p == r, tp_ref[r], acc)

        for s in range(3):
            idx = feat(1 + s).astype(i32)
            lo = jnp.bitwise_and(idx, LANES - 1)
            g0 = gather_row(ROW_STAT + 2 * s, lo)
            g1 = gather_row(ROW_STAT + 2 * s + 1, lo)
            acc = acc + jnp.where(idx < LANES, g0, g1)

        idx = feat(4).astype(i32)
        lo = jnp.bitwise_and(idx, LANES - 1)
        hi = jnp.right_shift(idx, 7)
        gp = gather_row(ROW_PLAT, lo)
        for c in range(1, 4):
            gp = jnp.where(hi == c, gather_row(ROW_PLAT + c, lo), gp)
        acc = acc + gp

        for k in range(N_CONT):
            acc = acc + wn_ref[k] * feat(5 + k)

        out_ref[g] = (jnp.transpose(acc, (1, 0))
                      + y_ref[g] * wy_ref[0] + by_ref[0])


def kernel(emb_period, emb_stations, emb_platforms, w_period, w_stations,
           w_platforms, w_fcn, w_fcy, b_fcy, x, y):
    if x.ndim == 1:
        x = x.reshape(1, -1)
    n = x.shape[0]
    bsg = 2
    blk = bsg * GROUP
    n_pad = ((n + blk - 1) // blk) * blk
    n_grp = n_pad // GROUP
    f32 = jnp.float32

    x = x.astype(f32)
    yv = y.reshape(n).astype(f32)
    if n_pad != n:
        x = jnp.pad(x, ((0, n_pad - n), (0, 0)))
        yv = jnp.pad(yv, (0, n_pad - n))
    x3 = x.reshape(n_grp, LANES, N_FEAT * J)
    y3 = yv.reshape(n_grp, LANES, J)

    tp = (emb_period @ w_period.T).reshape(N_PERIOD)
    ts = jnp.stack(
        [(emb_stations @ w_stations[:, 3 * c:3 * c + 3].T)[:, 0]
         for c in range(3)], axis=0)
    tpl = (emb_platforms @ w_platforms.T).reshape(N_PLATFORMS)

    tab = jnp.zeros((N_TAB_ROWS, LANES), f32)
    tab = tab.at[ROW_PLAT:ROW_PLAT + 4, :].set(
        jnp.pad(tpl, (0, 4 * LANES - N_PLATFORMS)).reshape(4, LANES))
    tab = tab.at[ROW_STAT:ROW_STAT + 6, :].set(
        jnp.pad(ts, ((0, 0), (0, 2 * LANES - N_STATIONS))).reshape(6, LANES))

    wn = w_fcn.reshape(N_CONT).astype(f32)
    wy = w_fcy.reshape(1).astype(f32)
    by = b_fcy.reshape(1).astype(f32)

    smem = pl.BlockSpec(memory_space=pltpu.MemorySpace.SMEM)
    grid = (n_pad // blk,)

    out = pl.pallas_call(
        _bl_kernel,
        out_shape=jax.ShapeDtypeStruct((n_grp, LANES, J), f32),
        grid=grid,
        in_specs=[
            pl.BlockSpec((bsg, LANES, N_FEAT * J), lambda i: (i, 0, 0)),
            pl.BlockSpec((bsg, LANES, J), lambda i: (i, 0, 0)),
            pl.BlockSpec((N_TAB_ROWS, LANES), lambda i: (0, 0)),
            smem,
            smem,
            smem,
            smem,
        ],
        out_specs=pl.BlockSpec((bsg, LANES, J), lambda i: (i, 0, 0)),
        scratch_shapes=[pltpu.VMEM((bsg * J * N_FEAT, LANES), f32)],
        compiler_params=pltpu.CompilerParams(
            dimension_semantics=("parallel",),
            vmem_limit_bytes=64 * 1024 * 1024),
    )(x3, y3, tab, tp.astype(f32), wn, wy, by)

    return out.reshape(-1)[:n].reshape(n, 1)


# speedup vs baseline: 1.2488x; 1.2000x over previous
import jax
import jax.numpy as jnp
from jax import lax
from jax.experimental import pallas as pl
from jax.experimental.pallas import tpu as pltpu

N_PERIOD = 4
N_STATIONS = 162
N_PLATFORMS = 460
N_CONT = 7
N_FEAT = 12
LANES = 128

ROW_PLAT = 0
ROW_STAT = 4
N_TAB_ROWS = 16


def _bl_kernel(x_ref, y_ref, tab_ref, eye_ref, tp_ref, wn_ref, wy_ref, by_ref,
               out_ref, scr):
    n_sub = x_ref.shape[0] // LANES
    i32 = jnp.int32
    f32 = jnp.float32

    def gather_row(r, idx):
        row = jnp.broadcast_to(tab_ref[r:r + 1, :], (8, LANES))
        return jnp.take_along_axis(row, idx, axis=1, mode="promise_in_bounds")

    for s in range(n_sub):
        slab = lax.dot_general(
            x_ref[s * LANES:(s + 1) * LANES, :], eye_ref[...],
            (((0,), (0,)), ((), ())), preferred_element_type=f32)
        scr[s * N_FEAT:(s + 1) * N_FEAT] = slab

    for g in range(n_sub // 8):
        base = g * 8 * N_FEAT

        def feat(k):
            return scr[base + k:base + 8 * N_FEAT:N_FEAT, :]

        idx_p = feat(0).astype(i32)
        acc = jnp.full((8, LANES), tp_ref[0], f32)
        for r in range(1, N_PERIOD):
            acc = jnp.where(idx_p == r, tp_ref[r], acc)

        for s in range(3):
            idx = feat(1 + s).astype(i32)
            lo = jnp.bitwise_and(idx, LANES - 1)
            g0 = gather_row(ROW_STAT + 2 * s, lo)
            g1 = gather_row(ROW_STAT + 2 * s + 1, lo)
            acc = acc + jnp.where(idx < LANES, g0, g1)

        idx = feat(4).astype(i32)
        lo = jnp.bitwise_and(idx, LANES - 1)
        hi = jnp.right_shift(idx, 7)
        gp = gather_row(ROW_PLAT, lo)
        for c in range(1, 4):
            gp = jnp.where(hi == c, gather_row(ROW_PLAT + c, lo), gp)
        acc = acc + gp

        for k in range(N_CONT):
            acc = acc + wn_ref[k] * feat(5 + k)

        out_ref[g * 8:(g + 1) * 8, :] = (acc + y_ref[g * 8:(g + 1) * 8, :]
                                         * wy_ref[0] + by_ref[0])


def kernel(emb_period, emb_stations, emb_platforms, w_period, w_stations,
           w_platforms, w_fcn, w_fcy, b_fcy, x, y):
    if x.ndim == 1:
        x = x.reshape(1, -1)
    n = x.shape[0]
    B = 8192
    n_pad = ((n + B - 1) // B) * B
    rows = n_pad // LANES
    f32 = jnp.float32

    x = x.astype(f32)
    yv = y.reshape(n).astype(f32)
    if n_pad != n:
        x = jnp.pad(x, ((0, n_pad - n), (0, 0)))
        yv = jnp.pad(yv, (0, n_pad - n))
    y2 = yv.reshape(rows, LANES)

    tp = (emb_period @ w_period.T).reshape(N_PERIOD)
    ts = jnp.stack(
        [(emb_stations @ w_stations[:, 3 * c:3 * c + 3].T)[:, 0]
         for c in range(3)], axis=0)
    tpl = (emb_platforms @ w_platforms.T).reshape(N_PLATFORMS)

    tab = jnp.zeros((N_TAB_ROWS, LANES), f32)
    tab = tab.at[ROW_PLAT:ROW_PLAT + 4, :].set(
        jnp.pad(tpl, (0, 4 * LANES - N_PLATFORMS)).reshape(4, LANES))
    tab = tab.at[ROW_STAT:ROW_STAT + 6, :].set(
        jnp.pad(ts, ((0, 0), (0, 2 * LANES - N_STATIONS))).reshape(6, LANES))

    eye = jnp.eye(LANES, dtype=f32)
    wn = w_fcn.reshape(N_CONT).astype(f32)
    wy = w_fcy.reshape(1).astype(f32)
    by = b_fcy.reshape(1).astype(f32)

    smem = pl.BlockSpec(memory_space=pltpu.MemorySpace.SMEM)
    grid = (n_pad // B,)
    rb = B // LANES

    out = pl.pallas_call(
        _bl_kernel,
        out_shape=jax.ShapeDtypeStruct((rows, LANES), f32),
        grid=grid,
        in_specs=[
            pl.BlockSpec((B, N_FEAT), lambda i: (i, 0)),
            pl.BlockSpec((rb, LANES), lambda i: (i, 0)),
            pl.BlockSpec((N_TAB_ROWS, LANES), lambda i: (0, 0)),
            pl.BlockSpec((LANES, LANES), lambda i: (0, 0)),
            smem,
            smem,
            smem,
            smem,
        ],
        out_specs=pl.BlockSpec((rb, LANES), lambda i: (i, 0)),
        scratch_shapes=[pltpu.VMEM((rb * N_FEAT, LANES), f32)],
        compiler_params=pltpu.CompilerParams(
            dimension_semantics=("parallel",),
            vmem_limit_bytes=64 * 1024 * 1024),
    )(x, y2, tab, eye, tp.astype(f32), wn, wy, by)

    return out.reshape(-1)[:n].reshape(n, 1)


# speedup vs baseline: 4.5046x; 3.6072x over previous
import jax
import jax.numpy as jnp
from jax.experimental import pallas as pl
from jax.experimental.pallas import tpu as pltpu

N_PERIOD = 4
N_STATIONS = 162
N_PLATFORMS = 460
N_CONT = 7
N_FEAT = 12
LANES = 128

ROW_PLAT = 0
ROW_STAT = 4
N_TAB_ROWS = 16


def _bl_kernel(x_ref, y_ref, tab_ref, tp_ref, wn_ref, wy_ref, by_ref,
               out_ref):
    bs = x_ref.shape[1]
    i32 = jnp.int32
    f32 = jnp.float32

    def gather_row(r, idx):
        row = jnp.broadcast_to(tab_ref[r:r + 1, :], (bs, LANES))
        return jnp.take_along_axis(row, idx, axis=1, mode="promise_in_bounds")

    idx_p = x_ref[0].astype(i32)
    acc = jnp.full((bs, LANES), tp_ref[0], f32)
    for r in range(1, N_PERIOD):
        acc = jnp.where(idx_p == r, tp_ref[r], acc)

    for s in range(3):
        idx = x_ref[1 + s].astype(i32)
        lo = jnp.bitwise_and(idx, LANES - 1)
        g0 = gather_row(ROW_STAT + 2 * s, lo)
        g1 = gather_row(ROW_STAT + 2 * s + 1, lo)
        acc = acc + jnp.where(idx < LANES, g0, g1)

    idx = x_ref[4].astype(i32)
    lo = jnp.bitwise_and(idx, LANES - 1)
    hi = jnp.right_shift(idx, 7)
    gp = gather_row(ROW_PLAT, lo)
    for c in range(1, 4):
        gp = jnp.where(hi == c, gather_row(ROW_PLAT + c, lo), gp)
    acc = acc + gp

    for k in range(N_CONT):
        acc = acc + wn_ref[k] * x_ref[5 + k]

    acc = acc + y_ref[...] * wy_ref[0] + by_ref[0]

    out_ref[...] = acc


def kernel(emb_period, emb_stations, emb_platforms, w_period, w_stations,
           w_platforms, w_fcn, w_fcy, b_fcy, x, y):
    if x.ndim == 1:
        x = x.reshape(1, -1)
    n = x.shape[0]
    bs = 512
    blk = bs * LANES
    n_pad = ((n + blk - 1) // blk) * blk
    rows = n_pad // LANES
    f32 = jnp.float32

    xt = jnp.pad(x.astype(f32).T, ((0, 0), (0, n_pad - n)))
    xt = xt.reshape(N_FEAT, rows, LANES)
    yt = jnp.pad(y.reshape(n).astype(f32), (0, n_pad - n)).reshape(rows, LANES)

    tp = (emb_period @ w_period.T).reshape(N_PERIOD)
    ts = jnp.stack(
        [(emb_stations @ w_stations[:, 3 * c:3 * c + 3].T)[:, 0]
         for c in range(3)], axis=0)
    tpl = (emb_platforms @ w_platforms.T).reshape(N_PLATFORMS)

    tab = jnp.zeros((N_TAB_ROWS, LANES), f32)
    tab = tab.at[ROW_PLAT:ROW_PLAT + 4, :].set(
        jnp.pad(tpl, (0, 4 * LANES - N_PLATFORMS)).reshape(4, LANES))
    tab = tab.at[ROW_STAT:ROW_STAT + 6, :].set(
        jnp.pad(ts, ((0, 0), (0, 2 * LANES - N_STATIONS))).reshape(6, LANES))

    wn = w_fcn.reshape(N_CONT).astype(f32)
    wy = w_fcy.reshape(1).astype(f32)
    by = b_fcy.reshape(1).astype(f32)

    smem = pl.BlockSpec(memory_space=pltpu.MemorySpace.SMEM)
    grid = (n_pad // blk,)

    out = pl.pallas_call(
        _bl_kernel,
        out_shape=jax.ShapeDtypeStruct((rows, LANES), f32),
        grid=grid,
        in_specs=[
            pl.BlockSpec((N_FEAT, bs, LANES), lambda i: (0, i, 0)),
            pl.BlockSpec((bs, LANES), lambda i: (i, 0)),
            pl.BlockSpec((N_TAB_ROWS, LANES), lambda i: (0, 0)),
            smem,
            smem,
            smem,
            smem,
        ],
        out_specs=pl.BlockSpec((bs, LANES), lambda i: (i, 0)),
        compiler_params=pltpu.CompilerParams(
            dimension_semantics=("parallel",),
            vmem_limit_bytes=64 * 1024 * 1024),
    )(xt, yt, tab, tp.astype(f32), wn, wy, by)

    return out.reshape(-1)[:n].reshape(n, 1)
